# baseline (speedup 1.0000x reference)
import functools

import jax
import jax.numpy as jnp
from jax.experimental import pallas as pl
from jax.experimental.pallas import tpu as pltpu

BLK = 2048
HLF = BLK // 2


def _half(x, w, b):
    P = w.shape[0]
    logits_t = jax.lax.dot_general(
        w, x, (((1,), (1,)), ((), ())),
        preferred_element_type=jnp.float32)
    logits_t = logits_t + b
    m = jnp.max(logits_t, axis=0, keepdims=True)
    sub = jax.lax.broadcasted_iota(jnp.int32, logits_t.shape, 0)
    sel = jnp.min(jnp.where(logits_t == m, sub, P), axis=0)
    sel = sel.astype(jnp.int32)
    mask_t = (sub == sel[None, :]).astype(jnp.float32)
    return sel, mask_t.T


def _router_kernel(x1_ref, x2_ref, w_ref, b_ref, sel_ref, mask_ref):
    w = w_ref[...]
    b = b_ref[...]
    sel1, mask1 = _half(x1_ref[...], w, b)
    sel2, mask2 = _half(x2_ref[...], w, b)
    mask_ref[0:HLF, :] = mask1
    mask_ref[HLF:BLK, :] = mask2
    sel_ref[0, 0, 0:HLF] = sel1
    sel_ref[0, 0, HLF:BLK] = sel2


@functools.partial(jax.jit, static_argnames=())
def kernel(process_feats, routing_matrix, bias):
    B, N, D = process_feats.shape
    P = routing_matrix.shape[0]
    T = B * N
    x = process_feats.reshape(T, D)
    b = bias.reshape(P, 1)
    grid = (T // BLK,)
    sel2d, mask = pl.pallas_call(
        _router_kernel,
        grid=grid,
        in_specs=[
            pl.BlockSpec((HLF, D), lambda i: (2 * i, 0)),
            pl.BlockSpec((HLF, D), lambda i: (2 * i + 1, 0)),
            pl.BlockSpec((P, D), lambda i: (0, 0)),
            pl.BlockSpec((P, 1), lambda i: (0, 0)),
        ],
        out_specs=[
            pl.BlockSpec((1, 1, BLK), lambda i: (i, 0, 0)),
            pl.BlockSpec((BLK, P), lambda i: (i, 0)),
        ],
        out_shape=[
            jax.ShapeDtypeStruct((T // BLK, 1, BLK), jnp.int32),
            jax.ShapeDtypeStruct((T, P), jnp.float32),
        ],
        compiler_params=pltpu.CompilerParams(
            dimension_semantics=("parallel",),
        ),
    )(x, x, routing_matrix, b)
    selected = sel2d.reshape(B, N)
    policy_mask = mask.reshape(B, N, P)
    return (selected, policy_mask)
